# baseline (speedup 1.0000x reference)
import functools

import jax
import jax.numpy as jnp
from jax import lax
from jax.experimental import pallas as pl
from jax.experimental.pallas import tpu as pltpu
from jax.experimental.pallas import tpu_sc as plsc

NC = 2
NS = 16

B = 10000

N_PAD = 327680
ROWS_BLK = 2048
CHUNK = 1024
HALF = 512
SCAT = 128
CP = 128

SEGS_PER_CORE = B // NC
ACC_ROWS = 5632
TRASH = 5120
OUT_TILE_ROWS = 1000


def _mlp_body(n_rows, x_ref, s_ref, w1_ref, w2_ref, w3_ref, o_ref):
    i = pl.program_id(0)
    bf = jnp.bfloat16
    h = jnp.dot(
        x_ref[...].astype(bf), w1_ref[...].astype(bf),
        preferred_element_type=jnp.float32,
    )
    h = jnp.maximum(h, 0.0)
    h = jnp.dot(
        h.astype(bf), w2_ref[...].astype(bf),
        preferred_element_type=jnp.float32,
    )
    h = jnp.maximum(h, 0.0)
    logits = jnp.dot(
        h.astype(bf), w3_ref[...].astype(bf),
        preferred_element_type=jnp.float32,
    )
    weighted = logits * s_ref[...]
    row = i * ROWS_BLK + lax.broadcasted_iota(jnp.int32, (ROWS_BLK, 1), 0)
    o_ref[...] = jnp.where(row < n_rows, weighted, 0.0)


def _slice_body(c_dim, p_ref, o_ref):
    o_ref[...] = p_ref[:, :c_dim]


def _sc_scatter(weighted, idx2, bounds):
    mesh = plsc.VectorSubcoreMesh(
        core_axis_name="c", subcore_axis_name="s", num_cores=NC, num_subcores=NS
    )
    n_chunks = N_PAD // CHUNK
    chunks_per_t = n_chunks // NS
    idx_rows = CHUNK // SCAT
    n_scat = HALF // SCAT

    @functools.partial(
        pl.kernel,
        mesh=mesh,
        out_type=jax.ShapeDtypeStruct((B, CP), jnp.float32),
        scratch_types=[
            pltpu.VMEM((idx_rows, SCAT), jnp.int32),
            pltpu.VMEM((HALF, CP), jnp.float32),
            pltpu.VMEM((80, SCAT), jnp.int32),
            pltpu.VMEM_SHARED((ACC_ROWS, CP), jnp.float32),
        ],
    )
    def scatter_k(wt_hbm, idx_hbm, bnd_hbm, z_hbm, out_hbm,
                  idx_v, rows_v, bnd_v, acc):
        cid = lax.axis_index("c")
        sid = lax.axis_index("s")
        seg_lo = cid * SEGS_PER_CORE
        seg_hi = seg_lo + SEGS_PER_CORE

        pltpu.sync_copy(bnd_hbm, bnd_v)

        @pl.when(sid == 0)
        def _():
            pltpu.sync_copy(z_hbm, acc)

        plsc.subcore_barrier()

        @pl.loop(0, chunks_per_t)
        def chunk_loop(i):
            c = sid + i * NS
            v_lo = bnd_v[c // 4, pl.ds((c % 4) * 32, 16)]
            v_hi = bnd_v[c // 4, pl.ds((c % 4) * 32 + 16, 16)]
            lo = v_lo[0]
            hi = v_hi[0]

            @pl.when((hi >= seg_lo) & (lo < seg_hi))
            def _():
                pltpu.sync_copy(idx_hbm.at[pl.ds(c * idx_rows, idx_rows)], idx_v)
                for r in range(idx_rows):
                    for k in range(SCAT // 16):
                        v = idx_v[r, pl.ds(k * 16, 16)]
                        inr = (v >= seg_lo) & (v < seg_hi)
                        idx_v[r, pl.ds(k * 16, 16)] = jnp.where(
                            inr, v - seg_lo, TRASH
                        )
                for h in range(CHUNK // HALF):
                    pltpu.sync_copy(
                        wt_hbm.at[pl.ds(c * CHUNK + h * HALF, HALF)], rows_v
                    )
                    for j in range(n_scat):
                        pltpu.sync_copy(
                            rows_v.at[pl.ds(j * SCAT, SCAT)],
                            acc.at[idx_v.at[h * n_scat + j]],
                            add=True,
                        )

        plsc.subcore_barrier()

        @pl.when(sid < SEGS_PER_CORE // OUT_TILE_ROWS)
        def _():
            pltpu.sync_copy(
                acc.at[pl.ds(sid * OUT_TILE_ROWS, OUT_TILE_ROWS)],
                out_hbm.at[
                    pl.ds(cid * SEGS_PER_CORE + sid * OUT_TILE_ROWS, OUT_TILE_ROWS)
                ],
            )

    zeros = jnp.zeros((ACC_ROWS, CP), jnp.float32)
    return scatter_k(weighted, idx2, bounds, zeros)


def kernel(X, ppr_scores, ppr_idx, W1, W2, W3):
    n, f = X.shape
    h_dim = W1.shape[1]
    c_dim = W3.shape[1]
    n_blocks = N_PAD // ROWS_BLK
    last_blk = (n - 1) // ROWS_BLK

    scores2 = ppr_scores.reshape(n, 1)
    w3p = jnp.pad(W3, ((0, 0), (0, CP - c_dim)))
    weighted = pl.pallas_call(
        functools.partial(_mlp_body, n),
        grid=(n_blocks,),
        in_specs=[
            pl.BlockSpec((ROWS_BLK, f), lambda i: (jnp.minimum(i, last_blk), 0)),
            pl.BlockSpec((ROWS_BLK, 1), lambda i: (jnp.minimum(i, last_blk), 0)),
            pl.BlockSpec((f, h_dim), lambda i: (0, 0)),
            pl.BlockSpec((h_dim, h_dim), lambda i: (0, 0)),
            pl.BlockSpec((h_dim, CP), lambda i: (0, 0)),
        ],
        out_specs=pl.BlockSpec((ROWS_BLK, CP), lambda i: (i, 0)),
        out_shape=jax.ShapeDtypeStruct((N_PAD, CP), jnp.float32),
    )(X, scores2, W1, W2, w3p)

    idxp = jnp.concatenate(
        [ppr_idx.astype(jnp.int32), jnp.full((N_PAD - n,), B - 1, jnp.int32)]
    )
    idx2 = idxp.reshape(N_PAD // SCAT, SCAT)
    n_chunks = N_PAD // CHUNK
    los = jnp.broadcast_to(idxp[::CHUNK, None], (n_chunks, 16))
    his = jnp.broadcast_to(idxp[CHUNK - 1 :: CHUNK, None], (n_chunks, 16))
    bounds = jnp.concatenate([los, his], axis=1).reshape(80, SCAT)
    acc128 = _sc_scatter(weighted, idx2, bounds)

    out = pl.pallas_call(
        functools.partial(_slice_body, c_dim),
        out_shape=jax.ShapeDtypeStruct((B, c_dim), jnp.float32),
    )(acc128)
    return out


# speedup vs baseline: 1.0276x; 1.0276x over previous
import functools

import jax
import jax.numpy as jnp
from jax import lax
from jax.experimental import pallas as pl
from jax.experimental.pallas import tpu as pltpu
from jax.experimental.pallas import tpu_sc as plsc

NC = 2
NS = 16

B = 10000

N_PAD = 327680
ROWS_BLK = 2048
CHUNK = 1024
HALF = 256
SCAT = 128
CP = 128

SEGS_PER_CORE = B // NC
ACC_ROWS = 5632
TRASH = 5120
OUT_TILE_ROWS = 1000


def _mlp_body(n_rows, x_ref, s_ref, w1_ref, w2_ref, w3_ref, o_ref):
    i = pl.program_id(0)
    bf = jnp.bfloat16
    h = jnp.dot(
        x_ref[...].astype(bf), w1_ref[...].astype(bf),
        preferred_element_type=jnp.float32,
    )
    h = jnp.maximum(h, 0.0)
    h = jnp.dot(
        h.astype(bf), w2_ref[...].astype(bf),
        preferred_element_type=jnp.float32,
    )
    h = jnp.maximum(h, 0.0)
    logits = jnp.dot(
        h.astype(bf), w3_ref[...].astype(bf),
        preferred_element_type=jnp.float32,
    )
    weighted = logits * s_ref[...]
    row = i * ROWS_BLK + lax.broadcasted_iota(jnp.int32, (ROWS_BLK, 1), 0)
    o_ref[...] = jnp.where(row < n_rows, weighted, 0.0)


def _slice_body(c_dim, p_ref, o_ref):
    o_ref[...] = p_ref[:, :c_dim]


def _sc_scatter(weighted, idx2, bounds):
    mesh = plsc.VectorSubcoreMesh(
        core_axis_name="c", subcore_axis_name="s", num_cores=NC, num_subcores=NS
    )
    n_chunks = N_PAD // CHUNK
    chunks_per_t = n_chunks // NS
    idx_rows = CHUNK // SCAT
    n_scat = HALF // SCAT

    @functools.partial(
        pl.kernel,
        mesh=mesh,
        out_type=jax.ShapeDtypeStruct((B, CP), jnp.float32),
        scratch_types=[
            pltpu.VMEM((idx_rows, SCAT), jnp.int32),
            pltpu.VMEM((HALF, CP), jnp.float32),
            pltpu.VMEM((HALF, CP), jnp.float32),
            pltpu.VMEM((80, SCAT), jnp.int32),
            pltpu.SemaphoreType.DMA,
            pltpu.SemaphoreType.DMA,
            pltpu.SemaphoreType.DMA,
            pltpu.SemaphoreType.DMA,
            pltpu.VMEM_SHARED((ACC_ROWS, CP), jnp.float32),
        ],
    )
    def scatter_k(wt_hbm, idx_hbm, bnd_hbm, z_hbm, out_hbm,
                  idx_v, rows_a, rows_b, bnd_v, sem_a, sem_b,
                  ssem_a, ssem_b, acc):
        cid = lax.axis_index("c")
        sid = lax.axis_index("s")
        seg_lo = cid * SEGS_PER_CORE
        seg_hi = seg_lo + SEGS_PER_CORE

        pltpu.sync_copy(bnd_hbm, bnd_v)

        @pl.when(sid == 0)
        def _():
            pltpu.sync_copy(z_hbm, acc)

        plsc.subcore_barrier()

        @pl.loop(0, chunks_per_t)
        def chunk_loop(i):
            c = sid + i * NS
            v_lo = bnd_v[c // 4, pl.ds((c % 4) * 32, 16)]
            v_hi = bnd_v[c // 4, pl.ds((c % 4) * 32 + 16, 16)]
            lo = v_lo[0]
            hi = v_hi[0]

            @pl.when((hi >= seg_lo) & (lo < seg_hi))
            def _():
                pltpu.sync_copy(idx_hbm.at[pl.ds(c * idx_rows, idx_rows)], idx_v)
                for r in range(idx_rows):
                    for k in range(SCAT // 16):
                        v = idx_v[r, pl.ds(k * 16, 16)]
                        inr = (v >= seg_lo) & (v < seg_hi)
                        idx_v[r, pl.ds(k * 16, 16)] = jnp.where(
                            inr, v - seg_lo, TRASH
                        )
                n_halves = CHUNK // HALF
                bufs = [rows_a, rows_b]
                sems = [sem_a, sem_b]
                ssems = [ssem_a, ssem_b]
                loads = [None] * n_halves
                scats = [[] for _ in range(n_halves)]
                loads[0] = pltpu.async_copy(
                    wt_hbm.at[pl.ds(c * CHUNK, HALF)], bufs[0], sems[0]
                )
                for h in range(n_halves):
                    buf, ssem = bufs[h % 2], ssems[h % 2]
                    if h + 1 < n_halves:
                        if h - 1 >= 0:
                            for d in scats[h - 1]:
                                d.wait()
                        loads[h + 1] = pltpu.async_copy(
                            wt_hbm.at[pl.ds(c * CHUNK + (h + 1) * HALF, HALF)],
                            bufs[(h + 1) % 2],
                            sems[(h + 1) % 2],
                        )
                    loads[h].wait()
                    for j in range(n_scat):
                        scats[h].append(
                            pltpu.async_copy(
                                buf.at[pl.ds(j * SCAT, SCAT)],
                                acc.at[idx_v.at[h * n_scat + j]],
                                ssem,
                                add=True,
                            )
                        )
                for d in scats[n_halves - 2] + scats[n_halves - 1]:
                    d.wait()

        plsc.subcore_barrier()

        @pl.when(sid < SEGS_PER_CORE // OUT_TILE_ROWS)
        def _():
            pltpu.sync_copy(
                acc.at[pl.ds(sid * OUT_TILE_ROWS, OUT_TILE_ROWS)],
                out_hbm.at[
                    pl.ds(cid * SEGS_PER_CORE + sid * OUT_TILE_ROWS, OUT_TILE_ROWS)
                ],
            )

    zeros = jnp.zeros((ACC_ROWS, CP), jnp.float32)
    return scatter_k(weighted, idx2, bounds, zeros)


def kernel(X, ppr_scores, ppr_idx, W1, W2, W3):
    n, f = X.shape
    h_dim = W1.shape[1]
    c_dim = W3.shape[1]
    n_blocks = N_PAD // ROWS_BLK
    last_blk = (n - 1) // ROWS_BLK

    scores2 = ppr_scores.reshape(n, 1)
    w3p = jnp.pad(W3, ((0, 0), (0, CP - c_dim)))
    weighted = pl.pallas_call(
        functools.partial(_mlp_body, n),
        grid=(n_blocks,),
        in_specs=[
            pl.BlockSpec((ROWS_BLK, f), lambda i: (jnp.minimum(i, last_blk), 0)),
            pl.BlockSpec((ROWS_BLK, 1), lambda i: (jnp.minimum(i, last_blk), 0)),
            pl.BlockSpec((f, h_dim), lambda i: (0, 0)),
            pl.BlockSpec((h_dim, h_dim), lambda i: (0, 0)),
            pl.BlockSpec((h_dim, CP), lambda i: (0, 0)),
        ],
        out_specs=pl.BlockSpec((ROWS_BLK, CP), lambda i: (i, 0)),
        out_shape=jax.ShapeDtypeStruct((N_PAD, CP), jnp.float32),
    )(X, scores2, W1, W2, w3p)

    idxp = jnp.concatenate(
        [ppr_idx.astype(jnp.int32), jnp.full((N_PAD - n,), B - 1, jnp.int32)]
    )
    idx2 = idxp.reshape(N_PAD // SCAT, SCAT)
    n_chunks = N_PAD // CHUNK
    los = jnp.broadcast_to(idxp[::CHUNK, None], (n_chunks, 16))
    his = jnp.broadcast_to(idxp[CHUNK - 1 :: CHUNK, None], (n_chunks, 16))
    bounds = jnp.concatenate([los, his], axis=1).reshape(80, SCAT)
    acc128 = _sc_scatter(weighted, idx2, bounds)

    out = pl.pallas_call(
        functools.partial(_slice_body, c_dim),
        out_shape=jax.ShapeDtypeStruct((B, c_dim), jnp.float32),
    )(acc128)
    return out
